# baseline (speedup 1.0000x reference)
import functools

import jax
import jax.numpy as jnp
from jax.experimental import pallas as pl
from jax.experimental.pallas import tpu as pltpu

_BM = 128


def _body(uv_ref, vu_ref, u0b_ref, u0sb_ref, i0b_ref, i0_ref,
          spec_i_ref, spec_u_ref, share_u_ref,
          uvb_scr, ab_scr, bbb_scr,
          *, bm, f):
    p = pl.program_id(0)
    s = pl.program_id(1)
    bf = jnp.bfloat16
    dims = (((1,), (0,)), ((), ()))
    rows = pl.ds(s * bm, bm)

    @pl.when(p == 0)
    def _phase0():
        uvb = uv_ref[...].astype(bf)
        uvb_scr[rows, :] = uvb
        a = jax.lax.dot_general(uvb, i0b_ref[...], dims,
                                preferred_element_type=jnp.float32)
        ab_scr[rows, :] = a.astype(bf)

    @pl.when(p == 1)
    def _phase1():
        vub = vu_ref[...].astype(bf)
        b = jax.lax.dot_general(vub, u0b_ref[...], dims,
                                preferred_element_type=jnp.float32)
        bs = jax.lax.dot_general(vub, u0sb_ref[...], dims,
                                 preferred_element_type=jnp.float32)
        d = jax.lax.dot_general(vub, ab_scr[...], dims,
                                preferred_element_type=jnp.float32)
        bbb_scr[rows, :f] = b.astype(bf)
        bbb_scr[rows, f:] = bs.astype(bf)
        spec_i_ref[...] = (i0_ref[...] + b + d) * (1.0 / 3.0)

    @pl.when(p == 2)
    def _phase2():
        uvb = uvb_scr[rows, :]
        acc = jax.lax.dot_general(uvb, bbb_scr[...], dims,
                                  preferred_element_type=jnp.float32)
        a = ab_scr[rows, :].astype(jnp.float32)
        u0 = u0b_ref[rows, :].astype(jnp.float32)
        u0s = u0sb_ref[rows, :].astype(jnp.float32)
        spec_u_ref[...] = (u0 + a + acc[:, :f]) * (1.0 / 3.0)
        share_u_ref[...] = (u0s + a + acc[:, f:]) * (1.0 / 3.0)


def _domain(UV, VU, u0, u0_share, i0):
    n_u, f = u0.shape
    n_i = i0.shape[0]
    bm = _BM
    ns = n_u // bm
    bf = jnp.bfloat16
    f32 = jnp.float32

    def stream0(p, s):
        return (jnp.where(p == 0, s, ns - 1), 0)

    def stream1(p, s):
        return (jnp.where(p == 1, s, jnp.where(p == 0, 0, ns - 1)), 0)

    def out2(p, s):
        return (jnp.where(p == 2, s, 0), 0)

    def const(p, s):
        return (0, 0)

    spec_i, spec_u, share_u = pl.pallas_call(
        functools.partial(_body, bm=bm, f=f),
        grid=(3, ns),
        in_specs=[
            pl.BlockSpec((bm, n_i), stream0),
            pl.BlockSpec((bm, n_u), stream1),
            pl.BlockSpec((n_u, f), const),
            pl.BlockSpec((n_u, f), const),
            pl.BlockSpec((n_i, f), const),
            pl.BlockSpec((bm, f), stream1),
        ],
        out_specs=(
            pl.BlockSpec((bm, f), stream1),
            pl.BlockSpec((bm, f), out2),
            pl.BlockSpec((bm, f), out2),
        ),
        out_shape=(
            jax.ShapeDtypeStruct((n_i, f), f32),
            jax.ShapeDtypeStruct((n_u, f), f32),
            jax.ShapeDtypeStruct((n_u, f), f32),
        ),
        scratch_shapes=[
            pltpu.VMEM((n_u, n_i), bf),
            pltpu.VMEM((n_u, f), bf),
            pltpu.VMEM((n_i, 2 * f), bf),
        ],
    )(UV, VU, u0.astype(bf), u0_share.astype(bf), i0.astype(bf), i0)

    return share_u, spec_u, spec_i


def kernel(source_UV, source_VU, target_UV, target_VU, source_user_emb,
           target_user_emb, source_item_emb, target_item_emb,
           source_user_emb_share, target_user_emb_share):
    s_share_u, s_spec_u, s_spec_i = _domain(
        source_UV, source_VU, source_user_emb, source_user_emb_share,
        source_item_emb)
    t_share_u, t_spec_u, t_spec_i = _domain(
        target_UV, target_VU, target_user_emb, target_user_emb_share,
        target_item_emb)
    return (s_share_u, s_spec_u, s_spec_i, t_share_u, t_spec_u, t_spec_i)


# speedup vs baseline: 1.2007x; 1.2007x over previous
import functools

import jax
import jax.numpy as jnp
from jax.experimental import pallas as pl
from jax.experimental.pallas import tpu as pltpu

_BM = 256
_CACHE_FRAC = 2


def _body(uv_ref, vu_ref, u0b_ref, u0sb_ref, i0b_ref, i0_ref,
          spec_i_ref, spec_u_ref, share_u_ref,
          uvb_scr, ab_scr, bbb_scr,
          *, bm, f):
    p = pl.program_id(0)
    s = pl.program_id(1)
    bf = jnp.bfloat16
    dims = (((1,), (0,)), ((), ()))
    rows = pl.ds(s * bm, bm)

    n_cached = uvb_scr.shape[0]
    cache_base = ab_scr.shape[0] - n_cached

    @pl.when(p == 0)
    def _phase0():
        uvb = uv_ref[...].astype(bf)

        @pl.when(s * bm >= cache_base)
        def _store():
            uvb_scr[pl.ds(s * bm - cache_base, bm), :] = uvb

        a = jax.lax.dot_general(uvb, i0b_ref[...], dims,
                                preferred_element_type=jnp.float32)
        ab_scr[rows, :] = a.astype(bf)

    @pl.when(p == 1)
    def _phase1():
        vub = vu_ref[...].astype(bf)
        b = jax.lax.dot_general(vub, u0b_ref[...], dims,
                                preferred_element_type=jnp.float32)
        bs = jax.lax.dot_general(vub, u0sb_ref[...], dims,
                                 preferred_element_type=jnp.float32)
        d = jax.lax.dot_general(vub, ab_scr[...], dims,
                                preferred_element_type=jnp.float32)
        bbb_scr[rows, :f] = b.astype(bf)
        bbb_scr[rows, f:] = bs.astype(bf)
        spec_i_ref[...] = (i0_ref[...] + b + d) * (1.0 / 3.0)

    def _phase2_combine(uvb):
        acc = jax.lax.dot_general(uvb, bbb_scr[...], dims,
                                  preferred_element_type=jnp.float32)
        a = ab_scr[rows, :].astype(jnp.float32)
        u0 = u0b_ref[rows, :].astype(jnp.float32)
        u0s = u0sb_ref[rows, :].astype(jnp.float32)
        spec_u_ref[...] = (u0 + a + acc[:, :f]) * (1.0 / 3.0)
        share_u_ref[...] = (u0s + a + acc[:, f:]) * (1.0 / 3.0)

    @pl.when(jnp.logical_and(p == 2, s * bm < cache_base))
    def _phase2_stream():
        _phase2_combine(uv_ref[...].astype(bf))

    @pl.when(jnp.logical_and(p == 2, s * bm >= cache_base))
    def _phase2_cached():
        _phase2_combine(uvb_scr[pl.ds(s * bm - cache_base, bm), :])


def _domain(UV, VU, u0, u0_share, i0):
    n_u, f = u0.shape
    n_i = i0.shape[0]
    bm = _BM
    ns = n_u // bm
    bf = jnp.bfloat16
    f32 = jnp.float32

    n_cached = n_u // _CACHE_FRAC
    cache_steps = (n_u - n_cached) // bm

    def stream0(p, s):
        return (jnp.where(p == 0, s,
                          jnp.where(p == 2, jnp.minimum(s, cache_steps - 1),
                                    ns - 1)), 0)

    def stream1(p, s):
        return (jnp.where(p == 1, s, jnp.where(p == 0, 0, ns - 1)), 0)

    def out2(p, s):
        return (jnp.where(p == 2, s, 0), 0)

    def const(p, s):
        return (0, 0)

    spec_i, spec_u, share_u = pl.pallas_call(
        functools.partial(_body, bm=bm, f=f),
        grid=(3, ns),
        in_specs=[
            pl.BlockSpec((bm, n_i), stream0),
            pl.BlockSpec((bm, n_u), stream1),
            pl.BlockSpec((n_u, f), const),
            pl.BlockSpec((n_u, f), const),
            pl.BlockSpec((n_i, f), const),
            pl.BlockSpec((bm, f), stream1),
        ],
        out_specs=(
            pl.BlockSpec((bm, f), stream1),
            pl.BlockSpec((bm, f), out2),
            pl.BlockSpec((bm, f), out2),
        ),
        out_shape=(
            jax.ShapeDtypeStruct((n_i, f), f32),
            jax.ShapeDtypeStruct((n_u, f), f32),
            jax.ShapeDtypeStruct((n_u, f), f32),
        ),
        scratch_shapes=[
            pltpu.VMEM((n_cached, n_i), bf),
            pltpu.VMEM((n_u, f), bf),
            pltpu.VMEM((n_i, 2 * f), bf),
        ],
    )(UV, VU, u0.astype(bf), u0_share.astype(bf), i0.astype(bf), i0)

    return share_u, spec_u, spec_i


def kernel(source_UV, source_VU, target_UV, target_VU, source_user_emb,
           target_user_emb, source_item_emb, target_item_emb,
           source_user_emb_share, target_user_emb_share):
    s_share_u, s_spec_u, s_spec_i = _domain(
        source_UV, source_VU, source_user_emb, source_user_emb_share,
        source_item_emb)
    t_share_u, t_spec_u, t_spec_i = _domain(
        target_UV, target_VU, target_user_emb, target_user_emb_share,
        target_item_emb)
    return (s_share_u, s_spec_u, s_spec_i, t_share_u, t_spec_u, t_spec_i)
